# baseline (speedup 1.0000x reference)
import jax
import jax.numpy as jnp
from jax import lax
from jax.experimental import pallas as pl
from jax.experimental.pallas import tpu as pltpu
from jax.experimental.pallas import tpu_sc as plsc

BATCH = 4096
HIST = 20
VOCAB = 1000
NC = 2
NS = 16
NW = NC * NS
PLANES_PER_WORKER = BATCH // NW
CHUNK_PLANES = 2
NCHUNKS = PLANES_PER_WORKER // CHUNK_PLANES
IDS_PER_CHUNK = CHUNK_PLANES * HIST
IDS_PER_WORKER = PLANES_PER_WORKER * HIST
NGROUPS = (IDS_PER_CHUNK + 15) // 16

def _sc_one_hot(arr_flat, out, ids_v, buf0, buf1, sem0, sem1):
    wid = lax.axis_index("s") * NC + lax.axis_index("c")
    base_plane = wid * PLANES_PER_WORKER
    ones = jnp.full((16,), 1.0, jnp.float32)
    zeros = jnp.zeros((16,), jnp.float32)
    bufs = (buf0, buf1)
    sems = (sem0, sem1)

    lane = lax.broadcasted_iota(jnp.int32, (16,), 0)
    r_pat, t_pat, m_pat = [], [], []
    for j in range(NGROUPS):
        lid = lane + j * 16
        rf = jnp.where(lid >= HIST, 1, 0) + jnp.where(lid >= 2 * HIST, 1, 0)
        r_pat.append(jnp.minimum(rf, CHUNK_PLANES - 1))
        t_pat.append(lid - rf * HIST)
        m_pat.append(lid < IDS_PER_CHUNK)

    pltpu.sync_copy(arr_flat.at[pl.ds(wid * IDS_PER_WORKER, IDS_PER_WORKER)],
                    ids_v.at[pl.ds(0, IDS_PER_WORKER)])
    ids_v[pl.ds(IDS_PER_WORKER, 16)] = jnp.zeros((16,), jnp.int32)

    def _zero_row(t, carry):
        for b in range(CHUNK_PLANES):
            for v0 in range(0, VOCAB - 15, 16):
                buf0[b, t, pl.ds(v0, 16)] = zeros
                buf1[b, t, pl.ds(v0, 16)] = zeros
            buf0[b, t, pl.ds(VOCAB - 16, 16)] = zeros
            buf1[b, t, pl.ds(VOCAB - 16, 16)] = zeros
        return carry

    lax.fori_loop(0, HIST, _zero_row, 0)

    def _scatter_chunk(buf, c, x):
        for j in range(NGROUPS):
            vids = ids_v[pl.ds(c * IDS_PER_CHUNK + j * 16, 16)]
            vids = jnp.minimum(vids, VOCAB - 1)
            plsc.store_scatter(buf, [r_pat[j], t_pat[j], vids], x,
                               mask=m_pat[j])

    def _dma(buf, c, sem):
        return pltpu.make_async_copy(
            buf,
            out.at[pl.ds(base_plane + c * CHUNK_PLANES, CHUNK_PLANES)],
            sem,
        )

    def _step(it, carry):
        for b in range(CHUNK_PLANES):
            c = it * CHUNK_PLANES + b

            @pl.when(c >= CHUNK_PLANES)
            def _reclaim():
                _dma(bufs[b], c - CHUNK_PLANES, sems[b]).wait()
                _scatter_chunk(bufs[b], c - CHUNK_PLANES, zeros)

            _scatter_chunk(bufs[b], c, ones)
            _dma(bufs[b], c, sems[b]).start()
        return carry

    lax.fori_loop(0, NCHUNKS // CHUNK_PLANES, _step, 0)

    for b in range(CHUNK_PLANES):
        _dma(bufs[b], NCHUNKS - CHUNK_PLANES + b, sems[b]).wait()


def kernel(arr, mask):
    del mask
    arr_flat = arr.reshape(-1).astype(jnp.int32)
    mesh = plsc.VectorSubcoreMesh(core_axis_name="c", subcore_axis_name="s",
                                  num_cores=NC, num_subcores=NS)
    f = pl.kernel(
        _sc_one_hot,
        out_type=jax.ShapeDtypeStruct((BATCH, HIST, VOCAB), jnp.float32),
        mesh=mesh,
        scratch_types=[
            pltpu.VMEM((IDS_PER_WORKER + 16,), jnp.int32),
            pltpu.VMEM((CHUNK_PLANES, HIST, VOCAB), jnp.float32),
            pltpu.VMEM((CHUNK_PLANES, HIST, VOCAB), jnp.float32),
            pltpu.SemaphoreType.DMA,
            pltpu.SemaphoreType.DMA,
        ],
        compiler_params=pltpu.CompilerParams(needs_layout_passes=False),
    )
    return f(arr_flat)
